# baseline (speedup 1.0000x reference)
import functools
import jax
import jax.numpy as jnp
from jax import lax
from jax.experimental import pallas as pl
from jax.experimental.pallas import tpu as pltpu
from jax.experimental.pallas import tpu_sc as plsc

M, B, D, C = 8, 16384, 128, 10
CP = 16
NEG = -3.0e38

NC, NS, L = 2, 16, 16
NW = NC * NS
S = B // NW


def _tc_votes_body(x_ref, w_ref, b_ref, votes_ref):
    bs = x_ref.shape[0]
    xT = x_ref[:].T
    logitsT = jnp.dot(w_ref[:], xT, preferred_element_type=jnp.float32)
    logitsT = logitsT + b_ref[:]
    l3 = logitsT.reshape(M, CP, bs)
    mx = jnp.max(l3, axis=1, keepdims=True)
    iota = lax.broadcasted_iota(jnp.int32, (M, CP, bs), 1)
    cand = jnp.where(l3 >= mx, iota, CP)
    votes_ref[:] = jnp.min(cand, axis=1)


def _sc_hist_body(votes_hbm, out_hbm, votes_v, counts_v):
    wid = lax.axis_index("s") * NC + lax.axis_index("c")
    base = wid * S
    pltpu.sync_copy(votes_hbm.at[:, pl.ds(base, S)], votes_v)

    ones = jnp.ones((L,), jnp.float32)
    zerosf = jnp.zeros((L,), jnp.float32)
    lane = lax.iota(jnp.int32, L)

    def hist_body(i, carry):
        sbase = i * L
        samp = lane + sbase
        for c in range(C):
            plsc.store_scatter(counts_v, [samp, jnp.full((L,), c, jnp.int32)],
                               zerosf)
        for m in range(M):
            v = votes_v[m, pl.ds(sbase, L)]
            plsc.addupdate_scatter(counts_v, [samp, v], ones)
        return carry

    lax.fori_loop(0, S // L, hist_body, 0)
    pltpu.sync_copy(counts_v, out_hbm.at[pl.ds(base, S)])


def kernel(x, W, b):
    W4 = jnp.zeros((M, CP, D), jnp.float32).at[:, :C, :].set(
        jnp.transpose(W, (0, 2, 1))).reshape(M * CP, D)
    b4 = jnp.full((M, CP), NEG, jnp.float32).at[:, :C].set(b)
    b4 = b4.reshape(M * CP, 1)
    bs = 1024
    votes = pl.pallas_call(
        _tc_votes_body,
        grid=(B // bs,),
        in_specs=[
            pl.BlockSpec((bs, D), lambda i: (i, 0)),
            pl.BlockSpec((M * CP, D), lambda i: (0, 0)),
            pl.BlockSpec((M * CP, 1), lambda i: (0, 0)),
        ],
        out_specs=pl.BlockSpec((M, bs), lambda i: (0, i)),
        out_shape=jax.ShapeDtypeStruct((M, B), jnp.int32),
    )(x, W4, b4)

    mesh = plsc.VectorSubcoreMesh(core_axis_name="c", subcore_axis_name="s")
    sc_hist = functools.partial(
        pl.kernel,
        mesh=mesh,
        compiler_params=pltpu.CompilerParams(needs_layout_passes=False),
        out_type=jax.ShapeDtypeStruct((B, C), jnp.float32),
        scratch_types=[
            pltpu.VMEM((M, S), jnp.int32),
            pltpu.VMEM((S, C), jnp.float32),
        ],
    )(_sc_hist_body)
    return sc_hist(votes)


# speedup vs baseline: 1.1329x; 1.1329x over previous
import functools
import jax
import jax.numpy as jnp
from jax import lax
from jax.experimental import pallas as pl
from jax.experimental.pallas import tpu as pltpu
from jax.experimental.pallas import tpu_sc as plsc

M, B, D, C = 8, 16384, 128, 10
CP = 16
NEG = -3.0e38

NC, NS, L = 2, 16, 16
NW = NC * NS
S = B // NW


def _tc_votes_body(x_ref, w_ref, b_ref, votes_ref):
    bs = x_ref.shape[0]
    xT = x_ref[:].T
    logitsT = jnp.dot(w_ref[:], xT, preferred_element_type=jnp.float32)
    logitsT = logitsT + b_ref[:]
    l3 = logitsT.reshape(M, CP, bs)
    mx = jnp.max(l3, axis=1, keepdims=True)
    iota = lax.broadcasted_iota(jnp.int32, (M, CP, bs), 1)
    cand = jnp.where(l3 >= mx, iota, CP)
    votes_ref[:] = jnp.min(cand, axis=1)


def _sc_hist_body(votes_hbm, out_hbm, votes_v, counts_v):
    wid = lax.axis_index("s") * NC + lax.axis_index("c")
    base = wid * S
    pltpu.sync_copy(votes_hbm.at[:, pl.ds(base, S)], votes_v)

    ones = jnp.ones((L,), jnp.float32)
    zerosf = jnp.zeros((L,), jnp.float32)
    lane = lax.iota(jnp.int32, L)

    def hist_body(i, carry):
        sbase = i * L
        samp = lane + sbase
        for c in range(C):
            plsc.store_scatter(counts_v, [samp, jnp.full((L,), c, jnp.int32)],
                               zerosf)
        for m in range(M):
            v = votes_v[m, pl.ds(sbase, L)]
            plsc.addupdate_scatter(counts_v, [samp, v], ones)
        return carry

    lax.fori_loop(0, S // L, hist_body, 0)
    pltpu.sync_copy(counts_v, out_hbm.at[pl.ds(base, S)])


def kernel(x, W, b):
    W4 = jnp.zeros((M, CP, D), jnp.float32).at[:, :C, :].set(
        jnp.transpose(W, (0, 2, 1))).reshape(M * CP, D)
    b4 = jnp.full((M, CP), NEG, jnp.float32).at[:, :C].set(b)
    b4 = b4.reshape(M * CP, 1)
    bs = 8192
    votes = pl.pallas_call(
        _tc_votes_body,
        grid=(B // bs,),
        in_specs=[
            pl.BlockSpec((bs, D), lambda i: (i, 0)),
            pl.BlockSpec((M * CP, D), lambda i: (0, 0)),
            pl.BlockSpec((M * CP, 1), lambda i: (0, 0)),
        ],
        out_specs=pl.BlockSpec((M, bs), lambda i: (0, i)),
        out_shape=jax.ShapeDtypeStruct((M, B), jnp.int32),
    )(x, W4, b4)

    mesh = plsc.VectorSubcoreMesh(core_axis_name="c", subcore_axis_name="s")
    sc_hist = functools.partial(
        pl.kernel,
        mesh=mesh,
        compiler_params=pltpu.CompilerParams(needs_layout_passes=False),
        out_type=jax.ShapeDtypeStruct((B, C), jnp.float32),
        scratch_types=[
            pltpu.VMEM((M, S), jnp.int32),
            pltpu.VMEM((S, C), jnp.float32),
        ],
    )(_sc_hist_body)
    return sc_hist(votes)


# speedup vs baseline: 1.1383x; 1.0048x over previous
import functools
import jax
import jax.numpy as jnp
from jax import lax
from jax.experimental import pallas as pl
from jax.experimental.pallas import tpu as pltpu
from jax.experimental.pallas import tpu_sc as plsc

M, B, D, C = 8, 16384, 128, 10
CP = 16
NEG = -3.0e38

NC, NS, L = 2, 16, 16
NW = NC * NS
S = B // NW


def _tc_votes_body(x_ref, w_ref, b_ref, votes_ref):
    bs = x_ref.shape[0]
    xT = x_ref[:].T
    logitsT = jnp.dot(w_ref[:], xT, preferred_element_type=jnp.float32)
    logitsT = logitsT + b_ref[:]
    l3 = logitsT.reshape(M, CP, bs)
    mx = jnp.max(l3, axis=1, keepdims=True)
    iota = lax.broadcasted_iota(jnp.int32, (M, CP, bs), 1)
    cand = jnp.where(l3 >= mx, iota, CP)
    votes_ref[:] = jnp.min(cand, axis=1)


def _sc_hist_body(votes_hbm, out_hbm, votes_v, counts_v):
    wid = lax.axis_index("s") * NC + lax.axis_index("c")
    base = wid * S
    pltpu.sync_copy(votes_hbm.at[:, pl.ds(base, S)], votes_v)

    ones = jnp.ones((L,), jnp.float32)
    zerosf = jnp.zeros((L,), jnp.float32)
    lane = lax.iota(jnp.int32, L)

    def hist_body(i, carry):
        sbase = i * L
        samp = lane + sbase
        for c in range(C):
            plsc.store_scatter(counts_v, [samp, jnp.full((L,), c, jnp.int32)],
                               zerosf)
        for m in range(M):
            v = votes_v[m, pl.ds(sbase, L)]
            plsc.addupdate_scatter(counts_v, [samp, v], ones)
        return carry

    lax.fori_loop(0, S // L, hist_body, 0)
    pltpu.sync_copy(counts_v, out_hbm.at[pl.ds(base, S)])


def kernel(x, W, b):
    W4 = jnp.zeros((M, CP, D), jnp.float32).at[:, :C, :].set(
        jnp.transpose(W, (0, 2, 1))).reshape(M * CP, D)
    b4 = jnp.full((M, CP), NEG, jnp.float32).at[:, :C].set(b)
    b4 = b4.reshape(M * CP, 1)
    bs = 4096
    votes = pl.pallas_call(
        _tc_votes_body,
        grid=(B // bs,),
        in_specs=[
            pl.BlockSpec((bs, D), lambda i: (i, 0)),
            pl.BlockSpec((M * CP, D), lambda i: (0, 0)),
            pl.BlockSpec((M * CP, 1), lambda i: (0, 0)),
        ],
        out_specs=pl.BlockSpec((M, bs), lambda i: (0, i)),
        out_shape=jax.ShapeDtypeStruct((M, B), jnp.int32),
    )(x, W4, b4)

    mesh = plsc.VectorSubcoreMesh(core_axis_name="c", subcore_axis_name="s")
    sc_hist = functools.partial(
        pl.kernel,
        mesh=mesh,
        compiler_params=pltpu.CompilerParams(
            needs_layout_passes=False, skip_device_barrier=True),
        out_type=jax.ShapeDtypeStruct((B, C), jnp.float32),
        scratch_types=[
            pltpu.VMEM((M, S), jnp.int32),
            pltpu.VMEM((S, C), jnp.float32),
        ],
    )(_sc_hist_body)
    return sc_hist(votes)
